# baseline (device time: 53516 ns/iter reference)
import jax
import jax.numpy as jnp
from jax import lax
from jax.experimental import pallas as pl
from jax.experimental.pallas import tpu as pltpu

N_DEV = 4
B = 2
SQ = 128
SKV = 128
D = 512
HQ = 8
DH = 64
SCALE = 0.125


def kernel(x, Wq, Wo, K_ext, V_ext):
    kv = jnp.concatenate(
        [K_ext.reshape(B * SKV, HQ * DH), V_ext.reshape(B * SKV, HQ * DH)],
        axis=0,
    )

    def body(x_ref, wq_ref, wo_ref, kv_ref, out_ref,
             q_ref, o_ref, comm_ref, send_sems, recv_sems):
        my = lax.axis_index("i")
        left = lax.rem(my + N_DEV - 1, N_DEV)
        right = lax.rem(my + 1, N_DEV)

        barrier = pltpu.get_barrier_semaphore()
        for nbr in (left, right):
            pl.semaphore_signal(
                barrier, inc=1,
                device_id=(nbr,), device_id_type=pl.DeviceIdType.MESH,
            )
        pl.semaphore_wait(barrier, 2)

        comm_ref[0] = kv_ref[...]

        for h in range(N_DEV - 1):
            rdma = pltpu.make_async_remote_copy(
                src_ref=comm_ref.at[h],
                dst_ref=comm_ref.at[h + 1],
                send_sem=send_sems.at[h],
                recv_sem=recv_sems.at[h + 1],
                device_id=(right,),
                device_id_type=pl.DeviceIdType.MESH,
            )
            rdma.start()
            rdma.wait()

        for b in range(B):
            q_ref[b] = jnp.dot(
                x_ref[b], wq_ref[...], preferred_element_type=jnp.float32
            )

        for b in range(B):
            for hd in range(HQ):
                q = q_ref[b, :, hd * DH:(hd + 1) * DH]
                s_parts = []
                for k in range(N_DEV):
                    kk = comm_ref[k, b * SKV:(b + 1) * SKV,
                                  hd * DH:(hd + 1) * DH]
                    s_parts.append(lax.dot_general(
                        q, kk, (((1,), (1,)), ((), ())),
                        preferred_element_type=jnp.float32,
                    ))
                s = jnp.concatenate(s_parts, axis=1) * SCALE
                m = jnp.max(s, axis=1, keepdims=True)
                p = jnp.exp(s - m)
                l = jnp.sum(p, axis=1, keepdims=True)
                acc = jnp.zeros((SQ, DH), jnp.float32)
                for k in range(N_DEV):
                    vv = comm_ref[k, B * SKV + b * SKV:B * SKV + (b + 1) * SKV,
                                  hd * DH:(hd + 1) * DH]
                    acc = acc + jnp.dot(
                        p[:, k * SKV:(k + 1) * SKV], vv,
                        preferred_element_type=jnp.float32,
                    )
                o_ref[b, :, hd * DH:(hd + 1) * DH] = acc / l

        for b in range(B):
            out_ref[b] = jnp.dot(
                o_ref[b], wo_ref[...], preferred_element_type=jnp.float32
            )

    return pl.pallas_call(
        body,
        out_shape=jax.ShapeDtypeStruct((B, SQ, D), jnp.float32),
        in_specs=[pl.BlockSpec(memory_space=pltpu.VMEM)] * 4,
        out_specs=pl.BlockSpec(memory_space=pltpu.VMEM),
        scratch_shapes=[
            pltpu.VMEM((B, SQ, HQ * DH), jnp.float32),
            pltpu.VMEM((B, SQ, HQ * DH), jnp.float32),
            pltpu.VMEM((N_DEV, 2 * B * SKV, HQ * DH), jnp.float32),
            pltpu.SemaphoreType.DMA((N_DEV,)),
            pltpu.SemaphoreType.DMA((N_DEV,)),
        ],
        compiler_params=pltpu.CompilerParams(collective_id=0),
    )(x, Wq, Wo, kv)


# device time: 49210 ns/iter; 1.0875x vs baseline; 1.0875x over previous
import jax
import jax.numpy as jnp
from jax import lax
from jax.experimental import pallas as pl
from jax.experimental.pallas import tpu as pltpu

N_DEV = 4
B = 2
SQ = 128
SKV = 128
D = 512
HQ = 8
DH = 64
SCALE = 0.125


def kernel(x, Wq, Wo, K_ext, V_ext):
    kv = jnp.concatenate(
        [K_ext.reshape(B * SKV, HQ * DH), V_ext.reshape(B * SKV, HQ * DH)],
        axis=0,
    )

    def body(x_ref, wq_ref, wo_ref, kv_ref, out_ref,
             q_ref, o_ref, m_ref, l_ref, comm_ref, send_sems, recv_sems):
        my = lax.axis_index("i")

        barrier = pltpu.get_barrier_semaphore()
        for off in (1, 2, 3):
            pl.semaphore_signal(
                barrier, inc=1,
                device_id=(lax.rem(my + off, N_DEV),),
                device_id_type=pl.DeviceIdType.MESH,
            )
        pl.semaphore_wait(barrier, 3)

        rdmas = {}
        for off in (1, 2, 3):
            rdmas[off] = pltpu.make_async_remote_copy(
                src_ref=kv_ref,
                dst_ref=comm_ref.at[N_DEV - off],
                send_sem=send_sems.at[off - 1],
                recv_sem=recv_sems.at[N_DEV - off],
                device_id=(lax.rem(my + off, N_DEV),),
                device_id_type=pl.DeviceIdType.MESH,
            )
            rdmas[off].start()

        for b in range(B):
            q_ref[b] = jnp.dot(
                x_ref[b], wq_ref[...], preferred_element_type=jnp.float32
            )

        def process_chunk(src, first):
            for b in range(B):
                for hd in range(HQ):
                    i = b * HQ + hd
                    cols = slice(hd * DH, (hd + 1) * DH)
                    q = q_ref[b, :, cols]
                    kk = src[b * SKV:(b + 1) * SKV, cols]
                    vv = src[(B + b) * SKV:(B + b + 1) * SKV, cols]
                    s = lax.dot_general(
                        q, kk, (((1,), (1,)), ((), ())),
                        preferred_element_type=jnp.float32,
                    ) * SCALE
                    mj = jnp.max(s, axis=1, keepdims=True)
                    if first:
                        p = jnp.exp(s - mj)
                        m_ref[i] = mj
                        l_ref[i] = jnp.sum(p, axis=1, keepdims=True)
                        o_ref[b, :, cols] = jnp.dot(
                            p, vv, preferred_element_type=jnp.float32
                        )
                    else:
                        m_old = m_ref[i]
                        m_new = jnp.maximum(m_old, mj)
                        alpha = jnp.exp(m_old - m_new)
                        p = jnp.exp(s - m_new)
                        m_ref[i] = m_new
                        l_ref[i] = l_ref[i] * alpha + jnp.sum(
                            p, axis=1, keepdims=True
                        )
                        o_ref[b, :, cols] = o_ref[b, :, cols] * alpha + jnp.dot(
                            p, vv, preferred_element_type=jnp.float32
                        )

        process_chunk(kv_ref, first=True)

        for j in (1, 3, 2):
            rdmas[N_DEV - j].wait_recv()
            process_chunk(comm_ref.at[j], first=False)

        for off in (1, 2, 3):
            rdmas[off].wait_send()

        for b in range(B):
            for hd in range(HQ):
                cols = slice(hd * DH, (hd + 1) * DH)
                o_ref[b, :, cols] = o_ref[b, :, cols] / l_ref[b * HQ + hd]
        for b in range(B):
            out_ref[b] = jnp.dot(
                o_ref[b], wo_ref[...], preferred_element_type=jnp.float32
            )

    return pl.pallas_call(
        body,
        out_shape=jax.ShapeDtypeStruct((B, SQ, D), jnp.float32),
        in_specs=[pl.BlockSpec(memory_space=pltpu.VMEM)] * 4,
        out_specs=pl.BlockSpec(memory_space=pltpu.VMEM),
        scratch_shapes=[
            pltpu.VMEM((B, SQ, HQ * DH), jnp.float32),
            pltpu.VMEM((B, SQ, HQ * DH), jnp.float32),
            pltpu.VMEM((B * HQ, SQ, 1), jnp.float32),
            pltpu.VMEM((B * HQ, SQ, 1), jnp.float32),
            pltpu.VMEM((N_DEV, 2 * B * SKV, HQ * DH), jnp.float32),
            pltpu.SemaphoreType.DMA((N_DEV - 1,)),
            pltpu.SemaphoreType.DMA((N_DEV,)),
        ],
        compiler_params=pltpu.CompilerParams(collective_id=0),
    )(x, Wq, Wo, kv)


# device time: 27108 ns/iter; 1.9742x vs baseline; 1.8153x over previous
import jax
import jax.numpy as jnp
from jax import lax
from jax.experimental import pallas as pl
from jax.experimental.pallas import tpu as pltpu

N_DEV = 4
B = 2
SQ = 128
SKV = 128
D = 512
HQ = 8
DH = 64
SCALE = 0.125

CDT = jnp.bfloat16


def kernel(x, Wq, Wo, K_ext, V_ext):
    kv = jnp.concatenate(
        [K_ext.reshape(B * SKV, HQ * DH), V_ext.reshape(B * SKV, HQ * DH)],
        axis=0,
    ).astype(CDT)
    Wq = (Wq * SCALE).astype(CDT)
    Wo = Wo.astype(CDT)
    x = x.astype(CDT)

    def body(x_ref, wq_ref, wo_ref, kv_ref, out_ref,
             q_ref, o_ref, s_ref, v_ref, comm_ref, send_sems, recv_sems):
        my = lax.axis_index("i")

        barrier = pltpu.get_barrier_semaphore()
        for off in (1, 2, 3):
            pl.semaphore_signal(
                barrier, inc=1,
                device_id=(lax.rem(my + off, N_DEV),),
                device_id_type=pl.DeviceIdType.MESH,
            )
        pl.semaphore_wait(barrier, 3)

        rdmas = {}
        for off in (1, 2, 3):
            rdmas[off] = pltpu.make_async_remote_copy(
                src_ref=kv_ref,
                dst_ref=comm_ref.at[N_DEV - off],
                send_sem=send_sems.at[off - 1],
                recv_sem=recv_sems.at[N_DEV - off],
                device_id=(lax.rem(my + off, N_DEV),),
                device_id_type=pl.DeviceIdType.MESH,
            )
            rdmas[off].start()

        for b in range(B):
            q_ref[b] = jnp.dot(
                x_ref[b], wq_ref[...], preferred_element_type=jnp.float32
            ).astype(CDT)

        def process_chunk(src, j):
            for b in range(B):
                v_ref[b, j * SKV:(j + 1) * SKV, :] = (
                    src[(B + b) * SKV:(B + b + 1) * SKV, :])
                for hd in range(HQ):
                    cols = slice(hd * DH, (hd + 1) * DH)
                    s_ref[b * HQ + hd, :, j * SKV:(j + 1) * SKV] = (
                        lax.dot_general(
                            q_ref[b, :, cols],
                            src[b * SKV:(b + 1) * SKV, cols],
                            (((1,), (1,)), ((), ())),
                            preferred_element_type=jnp.float32,
                        ))

        process_chunk(kv_ref, 0)
        for j in (1, 3, 2):
            rdmas[N_DEV - j].wait_recv()
            process_chunk(comm_ref.at[j], j)

        for off in (1, 2, 3):
            rdmas[off].wait_send()

        for b in range(B):
            for hd in range(HQ):
                cols = slice(hd * DH, (hd + 1) * DH)
                s = s_ref[b * HQ + hd]
                m = jnp.max(s, axis=1, keepdims=True)
                p = jnp.exp(s - m)
                l = jnp.sum(p, axis=1, keepdims=True)
                o_ref[b, :, cols] = (jnp.dot(
                    p.astype(CDT), v_ref[b, :, cols],
                    preferred_element_type=jnp.float32,
                ) / l).astype(CDT)
        for b in range(B):
            out_ref[b] = jnp.dot(
                o_ref[b], wo_ref[...], preferred_element_type=jnp.float32
            )

    return pl.pallas_call(
        body,
        out_shape=jax.ShapeDtypeStruct((B, SQ, D), jnp.float32),
        in_specs=[pl.BlockSpec(memory_space=pltpu.VMEM)] * 4,
        out_specs=pl.BlockSpec(memory_space=pltpu.VMEM),
        scratch_shapes=[
            pltpu.VMEM((B, SQ, HQ * DH), CDT),
            pltpu.VMEM((B, SQ, HQ * DH), CDT),
            pltpu.VMEM((B * HQ, SQ, N_DEV * SKV), jnp.float32),
            pltpu.VMEM((B, N_DEV * SKV, HQ * DH), CDT),
            pltpu.VMEM((N_DEV, 2 * B * SKV, HQ * DH), CDT),
            pltpu.SemaphoreType.DMA((N_DEV - 1,)),
            pltpu.SemaphoreType.DMA((N_DEV,)),
        ],
        compiler_params=pltpu.CompilerParams(collective_id=0),
    )(x, Wq, Wo, kv)


# device time: 24020 ns/iter; 2.2280x vs baseline; 1.1286x over previous
import jax
import jax.numpy as jnp
from jax import lax
from jax.experimental import pallas as pl
from jax.experimental.pallas import tpu as pltpu

N_DEV = 4
B = 2
SQ = 128
SKV = 128
D = 512
HQ = 8
DH = 64
SCALE = 0.125

CDT = jnp.bfloat16


def kernel(x, Wq, Wo, K_ext, V_ext):
    K2 = K_ext.reshape(B * SKV, HQ * DH)
    V2 = V_ext.reshape(B * SKV, HQ * DH)

    def body(x_ref, wq_ref, wo_ref, k_ref, v_in_ref, out_ref,
             q_ref, o_ref, s_ref, k_loc, k_comm, v_ref,
             ksend_sems, krecv_sems, vsend_sems, vrecv_sems):
        my = lax.axis_index("i")

        k_loc[...] = k_ref[...].astype(CDT)
        for b in range(B):
            v_ref[b, 0:SKV, :] = (
                v_in_ref[b * SKV:(b + 1) * SKV, :].astype(CDT))

        barrier = pltpu.get_barrier_semaphore()
        for off in (1, 2, 3):
            pl.semaphore_signal(
                barrier, inc=1,
                device_id=(lax.rem(my + off, N_DEV),),
                device_id_type=pl.DeviceIdType.MESH,
            )
        pl.semaphore_wait(barrier, 3)

        krdmas, vrdmas = {}, {}
        for off in (1, 2, 3):
            j = N_DEV - off
            peer = lax.rem(my + off, N_DEV)
            krdmas[j] = pltpu.make_async_remote_copy(
                src_ref=k_loc,
                dst_ref=k_comm.at[j],
                send_sem=ksend_sems.at[off - 1],
                recv_sem=krecv_sems.at[j],
                device_id=(peer,),
                device_id_type=pl.DeviceIdType.MESH,
            )
            krdmas[j].start()
        for off in (1, 2, 3):
            j = N_DEV - off
            peer = lax.rem(my + off, N_DEV)
            for b in range(B):
                vrdmas[(j, b)] = pltpu.make_async_remote_copy(
                    src_ref=v_ref.at[b, pl.ds(0, SKV), :],
                    dst_ref=v_ref.at[b, pl.ds(j * SKV, SKV), :],
                    send_sem=vsend_sems.at[off - 1, b],
                    recv_sem=vrecv_sems.at[j, b],
                    device_id=(peer,),
                    device_id_type=pl.DeviceIdType.MESH,
                )
                vrdmas[(j, b)].start()

        wq_bf = (wq_ref[...] * SCALE).astype(CDT)
        for b in range(B):
            q_ref[b] = jnp.dot(
                x_ref[b].astype(CDT), wq_bf,
                preferred_element_type=jnp.float32,
            ).astype(CDT)

        def s_blocks(src, j):
            for b in range(B):
                for hd in range(HQ):
                    cols = slice(hd * DH, (hd + 1) * DH)
                    s_ref[b * HQ + hd, :, j * SKV:(j + 1) * SKV] = (
                        lax.dot_general(
                            q_ref[b, :, cols],
                            src[b * SKV:(b + 1) * SKV, cols],
                            (((1,), (1,)), ((), ())),
                            preferred_element_type=jnp.float32,
                        ))

        s_blocks(k_loc, 0)
        for j in (1, 3, 2):
            krdmas[j].wait_recv()
            s_blocks(k_comm.at[j], j)

        for j in (1, 3, 2):
            for b in range(B):
                vrdmas[(j, b)].wait_recv()

        for j in (1, 2, 3):
            krdmas[j].wait_send()
            for b in range(B):
                vrdmas[(j, b)].wait_send()

        wo_bf = wo_ref[...].astype(CDT)
        for b in range(B):
            for hd in range(HQ):
                cols = slice(hd * DH, (hd + 1) * DH)
                s = s_ref[b * HQ + hd]
                m = jnp.max(s, axis=1, keepdims=True)
                p = jnp.exp(s - m)
                l = jnp.sum(p, axis=1, keepdims=True)
                o_ref[b, :, cols] = (jnp.dot(
                    p.astype(CDT), v_ref[b, :, cols],
                    preferred_element_type=jnp.float32,
                ) / l).astype(CDT)
        for b in range(B):
            out_ref[b] = jnp.dot(
                o_ref[b], wo_bf, preferred_element_type=jnp.float32
            )

    return pl.pallas_call(
        body,
        out_shape=jax.ShapeDtypeStruct((B, SQ, D), jnp.float32),
        in_specs=[pl.BlockSpec(memory_space=pltpu.VMEM)] * 5,
        out_specs=pl.BlockSpec(memory_space=pltpu.VMEM),
        scratch_shapes=[
            pltpu.VMEM((B, SQ, HQ * DH), CDT),
            pltpu.VMEM((B, SQ, HQ * DH), CDT),
            pltpu.VMEM((B * HQ, SQ, N_DEV * SKV), jnp.float32),
            pltpu.VMEM((B * SKV, HQ * DH), CDT),
            pltpu.VMEM((N_DEV, B * SKV, HQ * DH), CDT),
            pltpu.VMEM((B, N_DEV * SKV, HQ * DH), CDT),
            pltpu.SemaphoreType.DMA((N_DEV - 1,)),
            pltpu.SemaphoreType.DMA((N_DEV,)),
            pltpu.SemaphoreType.DMA((N_DEV - 1, B)),
            pltpu.SemaphoreType.DMA((N_DEV, B)),
        ],
        compiler_params=pltpu.CompilerParams(collective_id=0),
    )(x, Wq, Wo, K2, V2)


# device time: 22770 ns/iter; 2.3503x vs baseline; 1.0549x over previous
import jax
import jax.numpy as jnp
from jax import lax
from jax.experimental import pallas as pl
from jax.experimental.pallas import tpu as pltpu

N_DEV = 4
B = 2
SQ = 128
SKV = 128
D = 512
HQ = 8
DH = 64
SCALE = 0.125

CDT = jnp.bfloat16


def kernel(x, Wq, Wo, K_ext, V_ext):
    K2 = K_ext.reshape(B * SKV, HQ * DH)
    V2 = V_ext.reshape(B * SKV, HQ * DH)

    def body(x_ref, wq_ref, wo_ref, k_ref, v_in_ref, out_ref,
             q_ref, o_ref, s_ref, p_ref, l_ref, k_loc, k_comm, v_ref,
             ksend_sems, krecv_sems, vsend_sems, vrecv_sems):
        my = lax.axis_index("i")

        k_loc[...] = k_ref[...].astype(CDT)
        for b in range(B):
            v_ref[b, 0:SKV, :] = (
                v_in_ref[b * SKV:(b + 1) * SKV, :].astype(CDT))

        barrier = pltpu.get_barrier_semaphore()
        for off in (1, 2, 3):
            pl.semaphore_signal(
                barrier, inc=1,
                device_id=(lax.rem(my + off, N_DEV),),
                device_id_type=pl.DeviceIdType.MESH,
            )
        pl.semaphore_wait(barrier, 3)

        krdmas, vrdmas = {}, {}
        for off in (1, 2, 3):
            j = N_DEV - off
            peer = lax.rem(my + off, N_DEV)
            krdmas[j] = pltpu.make_async_remote_copy(
                src_ref=k_loc,
                dst_ref=k_comm.at[j],
                send_sem=ksend_sems.at[off - 1],
                recv_sem=krecv_sems.at[j],
                device_id=(peer,),
                device_id_type=pl.DeviceIdType.MESH,
            )
            krdmas[j].start()
        for off in (1, 2, 3):
            j = N_DEV - off
            peer = lax.rem(my + off, N_DEV)
            for b in range(B):
                vrdmas[(j, b)] = pltpu.make_async_remote_copy(
                    src_ref=v_ref.at[b, pl.ds(0, SKV), :],
                    dst_ref=v_ref.at[b, pl.ds(j * SKV, SKV), :],
                    send_sem=vsend_sems.at[off - 1, b],
                    recv_sem=vrecv_sems.at[j, b],
                    device_id=(peer,),
                    device_id_type=pl.DeviceIdType.MESH,
                )
                vrdmas[(j, b)].start()

        wq_bf = (wq_ref[...] * SCALE).astype(CDT)
        for b in range(B):
            q_ref[b] = jnp.dot(
                x_ref[b].astype(CDT), wq_bf,
                preferred_element_type=jnp.float32,
            ).astype(CDT)

        def s_blocks(src, j):
            for b in range(B):
                for hd in range(HQ):
                    cols = slice(hd * DH, (hd + 1) * DH)
                    s_ref[b * HQ + hd, :, j * SKV:(j + 1) * SKV] = (
                        lax.dot_general(
                            q_ref[b, :, cols],
                            src[b * SKV:(b + 1) * SKV, cols],
                            (((1,), (1,)), ((), ())),
                            preferred_element_type=jnp.float32,
                        ))

        s_blocks(k_loc, 0)
        for j in (1, 3, 2):
            krdmas[j].wait_recv()
            s_blocks(k_comm.at[j], j)

        for i in range(B * HQ):
            s = s_ref[i]
            m = jnp.max(s, axis=1, keepdims=True)
            p = jnp.exp(s - m)
            l_ref[i] = jnp.sum(p, axis=1, keepdims=True)
            p_ref[i] = p.astype(CDT)

        for j in (1, 3, 2):
            for b in range(B):
                vrdmas[(j, b)].wait_recv()

        wo_bf = wo_ref[...].astype(CDT)
        for b in range(B):
            for hd in range(HQ):
                cols = slice(hd * DH, (hd + 1) * DH)
                i = b * HQ + hd
                o_ref[b, :, cols] = (jnp.dot(
                    p_ref[i], v_ref[b, :, cols],
                    preferred_element_type=jnp.float32,
                ) / l_ref[i]).astype(CDT)
        for b in range(B):
            out_ref[b] = jnp.dot(
                o_ref[b], wo_bf, preferred_element_type=jnp.float32
            )

        for j in (1, 2, 3):
            krdmas[j].wait_send()
            for b in range(B):
                vrdmas[(j, b)].wait_send()

    return pl.pallas_call(
        body,
        out_shape=jax.ShapeDtypeStruct((B, SQ, D), jnp.float32),
        in_specs=[pl.BlockSpec(memory_space=pltpu.VMEM)] * 5,
        out_specs=pl.BlockSpec(memory_space=pltpu.VMEM),
        scratch_shapes=[
            pltpu.VMEM((B, SQ, HQ * DH), CDT),
            pltpu.VMEM((B, SQ, HQ * DH), CDT),
            pltpu.VMEM((B * HQ, SQ, N_DEV * SKV), jnp.float32),
            pltpu.VMEM((B * HQ, SQ, N_DEV * SKV), CDT),
            pltpu.VMEM((B * HQ, SQ, 1), jnp.float32),
            pltpu.VMEM((B * SKV, HQ * DH), CDT),
            pltpu.VMEM((N_DEV, B * SKV, HQ * DH), CDT),
            pltpu.VMEM((B, N_DEV * SKV, HQ * DH), CDT),
            pltpu.SemaphoreType.DMA((N_DEV - 1,)),
            pltpu.SemaphoreType.DMA((N_DEV,)),
            pltpu.SemaphoreType.DMA((N_DEV - 1, B)),
            pltpu.SemaphoreType.DMA((N_DEV, B)),
        ],
        compiler_params=pltpu.CompilerParams(collective_id=0),
    )(x, Wq, Wo, K2, V2)
